# baseline (speedup 1.0000x reference)
import functools
import jax
import jax.numpy as jnp
from jax import lax
from jax.experimental import pallas as pl
from jax.experimental.pallas import tpu as pltpu
from jax.experimental.pallas import tpu_sc as plsc

N = 10000
E = 80000
NUM_ET = 4
IN_C = 128
HID = 64
HEADS = 8
DH = HID // HEADS
OUT_C = 4
NUM_NT = 4

BLK = 1000


def _a_body(x_ref, nt_ref, Wx_ref, tb_ref, A_ref, tsrc_ref, tdst_ref):
    x = x_ref[...]
    nt = nt_ref[...]
    h = jnp.dot(x, Wx_ref[...], preferred_element_type=jnp.float32)
    tb = tb_ref[...]
    acc = jnp.zeros((BLK, HID), dtype=jnp.float32)
    for t in range(NUM_NT):
        acc = acc + jnp.where(nt == t, 1.0, 0.0) * tb[t][None, :]
    h = h + acc
    a = jnp.dot(h, A_ref[...], preferred_element_type=jnp.float32)
    for e in range(NUM_ET):
        tsrc_ref[e] = jnp.concatenate([h, a[:, e * 8:(e + 1) * 8]], axis=1)
        tdst_ref[e] = jnp.concatenate(
            [a[:, 32 + e * 8:32 + (e + 1) * 8],
             jnp.zeros((BLK, 8), dtype=jnp.float32)], axis=1)


def _phase_a(x, node_types, Wx, tb, A):
    nt2 = node_types.astype(jnp.int32).reshape(N, 1)
    return pl.pallas_call(
        _a_body,
        grid=(N // BLK,),
        in_specs=[
            pl.BlockSpec((BLK, IN_C), lambda i: (i, 0)),
            pl.BlockSpec((BLK, 1), lambda i: (i, 0)),
            pl.BlockSpec((IN_C, HID), lambda i: (0, 0)),
            pl.BlockSpec((NUM_NT, HID), lambda i: (0, 0)),
            pl.BlockSpec((HID, 64), lambda i: (0, 0)),
        ],
        out_specs=[
            pl.BlockSpec((NUM_ET, BLK, 72), lambda i: (0, i, 0)),
            pl.BlockSpec((NUM_ET, BLK, 16), lambda i: (0, i, 0)),
        ],
        out_shape=[
            jax.ShapeDtypeStruct((NUM_ET, N, 72), jnp.float32),
            jax.ShapeDtypeStruct((NUM_ET, N, 16), jnp.float32),
        ],
    )(x, nt2, Wx, tb, A)


CHUNK = 80
ZROWS = 80
NZ = N // ZROWS
MAXCH = 63
BIGN = MAXCH * CHUNK
NBUF = 3


def _sc_body(tsrc_hbm, tdst_hbm, edges_hbm, out_hbm,
             acc, z80, bigsrc, bigdst, gsrc, gdst, dscat, srcrows, dstrows,
             semb, sg0, sg1, sg2, ss0, ss1, ss2):
    c = lax.axis_index("c")
    s = lax.axis_index("s")

    zv = jnp.zeros((16,), jnp.float32)

    def zrow(r, _):
        for col in (0, 16, 32, 48, 56):
            z80[r, pl.ds(col, 16)] = zv
        return _
    lax.fori_loop(0, ZROWS, zrow, None)

    def zacc(jj, _):
        j = s + 16 * jj

        @pl.when(j < NZ)
        def _():
            for k in range(2):
                pltpu.sync_copy(z80, acc.at[k].at[pl.ds(j * ZROWS, ZROWS)])
        return _
    lax.fori_loop(0, (NZ + 15) // 16, zacc, None)
    plsc.subcore_barrier()

    lanes = lax.iota(jnp.int32, 16)
    cols8 = lanes & 7
    half = lanes >> 3
    sems = (sg0, sg1, sg2)
    ssems = (ss0, ss1, ss2)

    start = jnp.where(s < 8, 63 * s, 504 + 62 * (s - 8))
    nch = jnp.where(s < 8, MAXCH, 62)

    for k in range(2):
        et = c * 2 + k
        node_off = et * N
        sbase = (2 * et) * E + start * CHUNK
        dbase = (2 * et + 1) * E + start * CHUNK

        cb1 = pltpu.async_copy(edges_hbm.at[pl.ds(sbase, BIGN)], bigsrc, semb)
        cb2 = pltpu.async_copy(edges_hbm.at[pl.ds(dbase, BIGN)], bigdst, semb)
        cb1.wait()
        cb2.wait()

        def prep(i, b):
            o = i * CHUNK
            for t in range(CHUNK // 16):
                sl = pl.ds(16 * t, 16)
                sv = bigsrc[pl.ds(o + 16 * t, 16)]
                dv = bigdst[pl.ds(o + 16 * t, 16)]
                gsrc[b, sl] = sv + node_off
                gdst[b, sl] = dv + node_off
                dscat[b, sl] = dv

        def fire(b):
            pltpu.async_copy(tsrc_hbm.at[gsrc.at[b]], srcrows.at[b], sems[b])
            pltpu.async_copy(tdst_hbm.at[gdst.at[b]], dstrows.at[b], sems[b])

        def wait_g(b):
            pltpu.make_async_copy(tsrc_hbm.at[gsrc.at[b]], srcrows.at[b],
                                  sems[b]).wait()
            pltpu.make_async_copy(tdst_hbm.at[gdst.at[b]], dstrows.at[b],
                                  sems[b]).wait()

        def compute(b):
            srb = srcrows.at[b]
            drb = dstrows.at[b]

            @plsc.parallel_loop(0, CHUNK // 2, unroll=4)
            def _(p):
                rows2 = 2 * p + half
                a_s = plsc.load_gather(srb, [rows2, 64 + cols8])
                a_d = plsc.load_gather(drb, [rows2, cols8])
                alpha = a_s + a_d
                alpha = jnp.maximum(alpha, 0.2 * alpha)
                ex = jnp.exp(alpha)
                plsc.store_scatter(srb, [rows2, 64 + cols8], ex)
                for sub in range(2):
                    cc = 2 * p + sub
                    rows_cc = jnp.full((16,), 0, jnp.int32) + cc
                    for v in range(4):
                        bc = plsc.load_gather(srb, [rows_cc,
                                               (64 + 2 * v) + half])
                        sl = pl.ds(16 * v, 16)
                        srb[cc, sl] = srb[cc, sl] * bc

        def fire_scatter(b):
            pltpu.async_copy(srcrows.at[b], acc.at[k].at[dscat.at[b]],
                             ssems[b], add=True)

        def wait_scatter(b):
            pltpu.make_async_copy(srcrows.at[b], acc.at[k].at[dscat.at[b]],
                                  ssems[b]).wait()

        prep(0, 0)
        fire(0)
        prep(1, 1)
        fire(1)

        def tri_body(i3, _):
            for b in range(NBUF):
                i = NBUF * i3 + b
                b2 = (b + 2) % NBUF

                @pl.when(i < nch)
                def _():
                    wait_g(b)
                    compute(b)
                    fire_scatter(b)

                @pl.when((i >= 1) & (i <= nch))
                def _():
                    wait_scatter(b2)

                @pl.when(i + 2 < nch)
                def _():
                    prep(i + 2, b2)
                    fire(b2)
            return _
        lax.fori_loop(0, MAXCH // NBUF, tri_body, None)

        @pl.when(s < 8)
        def _():
            wait_scatter(62 % NBUF)

    plsc.subcore_barrier()

    for k in range(2):
        et = c * 2 + k

        def cp_body(jj, _):
            j = s + 16 * jj

            @pl.when(j < NZ)
            def _():
                pltpu.sync_copy(acc.at[k].at[pl.ds(j * ZROWS, ZROWS)],
                                out_hbm.at[pl.ds(et * N + j * ZROWS, ZROWS),
                                           pl.ds(0, 72)])
            return _
        lax.fori_loop(0, (NZ + 15) // 16, cp_body, None)


def _phase_b_sc(tsrc_flat, tdst_flat, edges_cat):
    mesh = plsc.VectorSubcoreMesh(core_axis_name="c", subcore_axis_name="s")
    f = pl.kernel(
        _sc_body,
        out_type=jax.ShapeDtypeStruct((NUM_ET * N, 128), jnp.float32),
        mesh=mesh,
        compiler_params=pltpu.CompilerParams(needs_layout_passes=False,
                                             use_tc_tiling_on_sc=False),
        scratch_types=[
            pltpu.VMEM_SHARED((2, N, 72), jnp.float32),
            pltpu.VMEM((ZROWS, 72), jnp.float32),
            pltpu.VMEM((BIGN,), jnp.int32),
            pltpu.VMEM((BIGN,), jnp.int32),
            pltpu.VMEM((NBUF, CHUNK), jnp.int32),
            pltpu.VMEM((NBUF, CHUNK), jnp.int32),
            pltpu.VMEM((NBUF, CHUNK), jnp.int32),
            pltpu.VMEM((NBUF, CHUNK, 72), jnp.float32),
            pltpu.VMEM((NBUF, CHUNK, 16), jnp.float32),
            pltpu.SemaphoreType.DMA,
            pltpu.SemaphoreType.DMA,
            pltpu.SemaphoreType.DMA,
            pltpu.SemaphoreType.DMA,
            pltpu.SemaphoreType.DMA,
            pltpu.SemaphoreType.DMA,
            pltpu.SemaphoreType.DMA,
        ],
    )
    return f(tsrc_flat, tdst_flat, edges_cat)


def _c1_body(a0_ref, a1_ref, a2_ref, a3_ref, K_ref, kb_ref, R_ref, S_ref):
    i = pl.program_id(0)
    parts = []
    for ar in (a0_ref, a1_ref, a2_ref, a3_ref):
        blk = ar[...]
        num = blk[:, 0:64]
        den = blk[:, 64:72]
        denb = jnp.dot(den, R_ref[...], preferred_element_type=jnp.float32)
        out_e = jnp.maximum(num / (denb + 1e-16), 0.0)
        t = jnp.tanh(jnp.dot(out_e, K_ref[...],
                             preferred_element_type=jnp.float32) + kb_ref[...])
        parts.append(jnp.sum(t, axis=0, keepdims=True))
    P = jnp.concatenate(parts, axis=0)

    @pl.when(i == 0)
    def _():
        S_ref[...] = P

    @pl.when(i > 0)
    def _():
        S_ref[...] = S_ref[...] + P


def _et_spec():
    nb = N // BLK
    return [pl.BlockSpec((BLK, 128), functools.partial(
        lambda k, i: (k * nb + i, 0), k)) for k in range(NUM_ET)]


def _phase_c1(agg_flat, K, kb, R):
    return pl.pallas_call(
        _c1_body,
        grid=(N // BLK,),
        in_specs=_et_spec() + [
            pl.BlockSpec((HID, HID), lambda i: (0, 0)),
            pl.BlockSpec((1, HID), lambda i: (0, 0)),
            pl.BlockSpec((8, HID), lambda i: (0, 0)),
        ],
        out_specs=pl.BlockSpec((NUM_ET, HID), lambda i: (0, 0)),
        out_shape=jax.ShapeDtypeStruct((NUM_ET, HID), jnp.float32),
    )(agg_flat, agg_flat, agg_flat, agg_flat, K, kb, R)


def _c2_body(a0_ref, a1_ref, a2_ref, a3_ref, S_ref, q_ref, R_ref, lw_ref,
             lb_ref, out_ref):
    score = jnp.dot(S_ref[...], q_ref[...],
                    preferred_element_type=jnp.float32) / N
    m = jnp.max(score)
    b = jnp.exp(score - m)
    beta = b / jnp.sum(b)
    sem = jnp.zeros((BLK, HID), dtype=jnp.float32)
    for e, ar in enumerate((a0_ref, a1_ref, a2_ref, a3_ref)):
        blk = ar[...]
        num = blk[:, 0:64]
        den = blk[:, 64:72]
        denb = jnp.dot(den, R_ref[...], preferred_element_type=jnp.float32)
        out_e = jnp.maximum(num / (denb + 1e-16), 0.0)
        sem = sem + beta[e, 0] * out_e
    out_ref[...] = jnp.dot(jnp.maximum(sem, 0.0), lw_ref[...],
                           preferred_element_type=jnp.float32) + lb_ref[...]


def _phase_c2(agg_flat, S, qv, R, lw, lb):
    return pl.pallas_call(
        _c2_body,
        grid=(N // BLK,),
        in_specs=_et_spec() + [
            pl.BlockSpec((NUM_ET, HID), lambda i: (0, 0)),
            pl.BlockSpec((HID, 1), lambda i: (0, 0)),
            pl.BlockSpec((8, HID), lambda i: (0, 0)),
            pl.BlockSpec((HID, OUT_C), lambda i: (0, 0)),
            pl.BlockSpec((1, OUT_C), lambda i: (0, 0)),
        ],
        out_specs=pl.BlockSpec((BLK, OUT_C), lambda i: (i, 0)),
        out_shape=jax.ShapeDtypeStruct((N, OUT_C), jnp.float32),
    )(agg_flat, agg_flat, agg_flat, agg_flat, S, qv, R, lw, lb)


def kernel(x, node_types, edge_index_0, edge_index_1, edge_index_2,
           edge_index_3, type_emb, proj_W, proj_b, att_src, att_dst,
           k_lin_W, k_lin_b, q, lin_W, lin_b):
    Wx = proj_W[:IN_C]
    tb = type_emb @ proj_W[IN_C:] + proj_b
    eye8 = jnp.eye(8, dtype=jnp.float32)
    As = jnp.einsum('ehj,hk->hjek', att_src, eye8).reshape(HID, 32)
    Ad = jnp.einsum('ehj,hk->hjek', att_dst, eye8).reshape(HID, 32)
    A = jnp.concatenate([As, Ad], axis=1)
    R = jnp.repeat(eye8, 8, axis=1)

    tsrc, tdst = _phase_a(x, node_types, Wx, tb, A)
    tsrc_flat = tsrc.reshape(NUM_ET * N, 72)
    tdst_flat = tdst.reshape(NUM_ET * N, 16)

    eis = (edge_index_0, edge_index_1, edge_index_2, edge_index_3)
    pad = jnp.zeros((CHUNK,), jnp.int32)
    edges_cat = jnp.concatenate(
        [ei.astype(jnp.int32).reshape(-1) for ei in eis] + [pad])
    agg_flat = _phase_b_sc(tsrc_flat, tdst_flat, edges_cat)

    S = _phase_c1(agg_flat, k_lin_W, k_lin_b.reshape(1, HID), R)
    out = _phase_c2(agg_flat, S, q.reshape(HID, 1), R, lin_W,
                    lin_b.reshape(1, OUT_C))
    return out


# speedup vs baseline: 1.0350x; 1.0350x over previous
import functools
import jax
import jax.numpy as jnp
from jax import lax
from jax.experimental import pallas as pl
from jax.experimental.pallas import tpu as pltpu
from jax.experimental.pallas import tpu_sc as plsc

N = 10000
E = 80000
NUM_ET = 4
IN_C = 128
HID = 64
HEADS = 8
DH = HID // HEADS
OUT_C = 4
NUM_NT = 4

BLK = 1000


def _a_body(x_ref, nt_ref, Wx_ref, tb_ref, A_ref, tsrc_ref, tdst_ref):
    x = x_ref[...]
    nt = nt_ref[...]
    h = jnp.dot(x, Wx_ref[...], preferred_element_type=jnp.float32)
    tb = tb_ref[...]
    acc = jnp.zeros((BLK, HID), dtype=jnp.float32)
    for t in range(NUM_NT):
        acc = acc + jnp.where(nt == t, 1.0, 0.0) * tb[t][None, :]
    h = h + acc
    a = jnp.dot(h, A_ref[...], preferred_element_type=jnp.float32)
    for e in range(NUM_ET):
        tsrc_ref[e] = jnp.concatenate([h, a[:, e * 8:(e + 1) * 8]], axis=1)
        tdst_ref[e] = jnp.concatenate(
            [a[:, 32 + e * 8:32 + (e + 1) * 8],
             jnp.zeros((BLK, 8), dtype=jnp.float32)], axis=1)


def _phase_a(x, node_types, Wx, tb, A):
    nt2 = node_types.astype(jnp.int32).reshape(N, 1)
    return pl.pallas_call(
        _a_body,
        grid=(N // BLK,),
        in_specs=[
            pl.BlockSpec((BLK, IN_C), lambda i: (i, 0)),
            pl.BlockSpec((BLK, 1), lambda i: (i, 0)),
            pl.BlockSpec((IN_C, HID), lambda i: (0, 0)),
            pl.BlockSpec((NUM_NT, HID), lambda i: (0, 0)),
            pl.BlockSpec((HID, 64), lambda i: (0, 0)),
        ],
        out_specs=[
            pl.BlockSpec((NUM_ET, BLK, 72), lambda i: (0, i, 0)),
            pl.BlockSpec((NUM_ET, BLK, 16), lambda i: (0, i, 0)),
        ],
        out_shape=[
            jax.ShapeDtypeStruct((NUM_ET, N, 72), jnp.float32),
            jax.ShapeDtypeStruct((NUM_ET, N, 16), jnp.float32),
        ],
    )(x, nt2, Wx, tb, A)


CHUNK = 80
ZROWS = 80
NZ = N // ZROWS
MAXCH = 63
BIGN = MAXCH * CHUNK
NBUF = 3


def _sc_body(tsrc_hbm, tdst_hbm, edges_hbm, out_hbm,
             acc, z80, bigsrc, bigdst, gsrc, gdst, dscat, srcrows, dstrows,
             semb, sg0, sg1, sg2, ss0, ss1, ss2):
    c = lax.axis_index("c")
    s = lax.axis_index("s")

    zv = jnp.zeros((16,), jnp.float32)

    def zrow(r, _):
        for col in (0, 16, 32, 48, 56):
            z80[r, pl.ds(col, 16)] = zv
        return _
    lax.fori_loop(0, ZROWS, zrow, None)

    def zacc(jj, _):
        j = s + 16 * jj

        @pl.when(j < NZ)
        def _():
            for k in range(2):
                pltpu.sync_copy(z80, acc.at[k].at[pl.ds(j * ZROWS, ZROWS)])
        return _
    lax.fori_loop(0, (NZ + 15) // 16, zacc, None)
    plsc.subcore_barrier()

    lanes = lax.iota(jnp.int32, 16)
    cols8 = lanes & 7
    half = lanes >> 3
    sems = (sg0, sg1, sg2)
    ssems = (ss0, ss1, ss2)

    start = jnp.where(s < 8, 63 * s, 504 + 62 * (s - 8))
    nch = jnp.where(s < 8, MAXCH, 62)

    for k in range(2):
        et = c * 2 + k
        node_off = et * N
        sbase = (2 * et) * E + start * CHUNK
        dbase = (2 * et + 1) * E + start * CHUNK

        cb1 = pltpu.async_copy(edges_hbm.at[pl.ds(sbase, BIGN)], bigsrc, semb)
        cb2 = pltpu.async_copy(edges_hbm.at[pl.ds(dbase, BIGN)], bigdst, semb)
        cb1.wait()
        cb2.wait()

        def prep(i, b):
            o = i * CHUNK
            for t in range(CHUNK // 16):
                sl = pl.ds(16 * t, 16)
                sv = bigsrc[pl.ds(o + 16 * t, 16)]
                dv = bigdst[pl.ds(o + 16 * t, 16)]
                gsrc[b, sl] = sv + node_off
                gdst[b, sl] = dv + node_off
                dscat[b, sl] = dv

        def fire(b):
            pltpu.async_copy(tsrc_hbm.at[gsrc.at[b]], srcrows.at[b], sems[b])
            pltpu.async_copy(tdst_hbm.at[gdst.at[b]], dstrows.at[b], sems[b])

        def wait_g(b):
            pltpu.make_async_copy(tsrc_hbm.at[gsrc.at[b]], srcrows.at[b],
                                  sems[b]).wait()
            pltpu.make_async_copy(tdst_hbm.at[gdst.at[b]], dstrows.at[b],
                                  sems[b]).wait()

        def compute(b):
            srb = srcrows.at[b]
            drb = dstrows.at[b]

            @plsc.parallel_loop(0, CHUNK // 2, unroll=2)
            def _(p):
                rows2 = 2 * p + half
                a_s = plsc.load_gather(srb, [rows2, 64 + cols8])
                a_d = plsc.load_gather(drb, [rows2, cols8])
                alpha = a_s + a_d
                alpha = jnp.maximum(alpha, 0.2 * alpha)
                ex = jnp.exp(alpha)
                plsc.store_scatter(srb, [rows2, 64 + cols8], ex)
                for sub in range(2):
                    cc = 2 * p + sub
                    rows_cc = jnp.full((16,), 0, jnp.int32) + cc
                    for v in range(4):
                        bc = plsc.load_gather(srb, [rows_cc,
                                               (64 + 2 * v) + half])
                        sl = pl.ds(16 * v, 16)
                        srb[cc, sl] = srb[cc, sl] * bc

        def fire_scatter(b):
            pltpu.async_copy(srcrows.at[b], acc.at[k].at[dscat.at[b]],
                             ssems[b], add=True)

        def wait_scatter(b):
            pltpu.make_async_copy(srcrows.at[b], acc.at[k].at[dscat.at[b]],
                                  ssems[b]).wait()

        prep(0, 0)
        fire(0)
        prep(1, 1)
        fire(1)

        def tri_body(i3, _):
            for b in range(NBUF):
                i = NBUF * i3 + b
                b2 = (b + 2) % NBUF

                @pl.when(i < nch)
                def _():
                    wait_g(b)
                    compute(b)
                    fire_scatter(b)

                @pl.when((i >= 1) & (i <= nch))
                def _():
                    wait_scatter(b2)

                @pl.when(i + 2 < nch)
                def _():
                    prep(i + 2, b2)
                    fire(b2)
            return _
        lax.fori_loop(0, MAXCH // NBUF, tri_body, None)

        @pl.when(s < 8)
        def _():
            wait_scatter(62 % NBUF)

    plsc.subcore_barrier()

    for k in range(2):
        et = c * 2 + k

        def cp_body(jj, _):
            j = s + 16 * jj

            @pl.when(j < NZ)
            def _():
                pltpu.sync_copy(acc.at[k].at[pl.ds(j * ZROWS, ZROWS)],
                                out_hbm.at[pl.ds(et * N + j * ZROWS, ZROWS),
                                           pl.ds(0, 72)])
            return _
        lax.fori_loop(0, (NZ + 15) // 16, cp_body, None)


def _phase_b_sc(tsrc_flat, tdst_flat, edges_cat):
    mesh = plsc.VectorSubcoreMesh(core_axis_name="c", subcore_axis_name="s")
    f = pl.kernel(
        _sc_body,
        out_type=jax.ShapeDtypeStruct((NUM_ET * N, 128), jnp.float32),
        mesh=mesh,
        compiler_params=pltpu.CompilerParams(needs_layout_passes=False,
                                             use_tc_tiling_on_sc=False),
        scratch_types=[
            pltpu.VMEM_SHARED((2, N, 72), jnp.float32),
            pltpu.VMEM((ZROWS, 72), jnp.float32),
            pltpu.VMEM((BIGN,), jnp.int32),
            pltpu.VMEM((BIGN,), jnp.int32),
            pltpu.VMEM((NBUF, CHUNK), jnp.int32),
            pltpu.VMEM((NBUF, CHUNK), jnp.int32),
            pltpu.VMEM((NBUF, CHUNK), jnp.int32),
            pltpu.VMEM((NBUF, CHUNK, 72), jnp.float32),
            pltpu.VMEM((NBUF, CHUNK, 16), jnp.float32),
            pltpu.SemaphoreType.DMA,
            pltpu.SemaphoreType.DMA,
            pltpu.SemaphoreType.DMA,
            pltpu.SemaphoreType.DMA,
            pltpu.SemaphoreType.DMA,
            pltpu.SemaphoreType.DMA,
            pltpu.SemaphoreType.DMA,
        ],
    )
    return f(tsrc_flat, tdst_flat, edges_cat)


def _c1_body(a0_ref, a1_ref, a2_ref, a3_ref, K_ref, kb_ref, R_ref, S_ref):
    i = pl.program_id(0)
    parts = []
    for ar in (a0_ref, a1_ref, a2_ref, a3_ref):
        blk = ar[...]
        num = blk[:, 0:64]
        den = blk[:, 64:72]
        denb = jnp.dot(den, R_ref[...], preferred_element_type=jnp.float32)
        out_e = jnp.maximum(num / (denb + 1e-16), 0.0)
        t = jnp.tanh(jnp.dot(out_e, K_ref[...],
                             preferred_element_type=jnp.float32) + kb_ref[...])
        parts.append(jnp.sum(t, axis=0, keepdims=True))
    P = jnp.concatenate(parts, axis=0)

    @pl.when(i == 0)
    def _():
        S_ref[...] = P

    @pl.when(i > 0)
    def _():
        S_ref[...] = S_ref[...] + P


def _et_spec():
    nb = N // BLK
    return [pl.BlockSpec((BLK, 128), functools.partial(
        lambda k, i: (k * nb + i, 0), k)) for k in range(NUM_ET)]


def _phase_c1(agg_flat, K, kb, R):
    return pl.pallas_call(
        _c1_body,
        grid=(N // BLK,),
        in_specs=_et_spec() + [
            pl.BlockSpec((HID, HID), lambda i: (0, 0)),
            pl.BlockSpec((1, HID), lambda i: (0, 0)),
            pl.BlockSpec((8, HID), lambda i: (0, 0)),
        ],
        out_specs=pl.BlockSpec((NUM_ET, HID), lambda i: (0, 0)),
        out_shape=jax.ShapeDtypeStruct((NUM_ET, HID), jnp.float32),
    )(agg_flat, agg_flat, agg_flat, agg_flat, K, kb, R)


def _c2_body(a0_ref, a1_ref, a2_ref, a3_ref, S_ref, q_ref, R_ref, lw_ref,
             lb_ref, out_ref):
    score = jnp.dot(S_ref[...], q_ref[...],
                    preferred_element_type=jnp.float32) / N
    m = jnp.max(score)
    b = jnp.exp(score - m)
    beta = b / jnp.sum(b)
    sem = jnp.zeros((BLK, HID), dtype=jnp.float32)
    for e, ar in enumerate((a0_ref, a1_ref, a2_ref, a3_ref)):
        blk = ar[...]
        num = blk[:, 0:64]
        den = blk[:, 64:72]
        denb = jnp.dot(den, R_ref[...], preferred_element_type=jnp.float32)
        out_e = jnp.maximum(num / (denb + 1e-16), 0.0)
        sem = sem + beta[e, 0] * out_e
    out_ref[...] = jnp.dot(jnp.maximum(sem, 0.0), lw_ref[...],
                           preferred_element_type=jnp.float32) + lb_ref[...]


def _phase_c2(agg_flat, S, qv, R, lw, lb):
    return pl.pallas_call(
        _c2_body,
        grid=(N // BLK,),
        in_specs=_et_spec() + [
            pl.BlockSpec((NUM_ET, HID), lambda i: (0, 0)),
            pl.BlockSpec((HID, 1), lambda i: (0, 0)),
            pl.BlockSpec((8, HID), lambda i: (0, 0)),
            pl.BlockSpec((HID, OUT_C), lambda i: (0, 0)),
            pl.BlockSpec((1, OUT_C), lambda i: (0, 0)),
        ],
        out_specs=pl.BlockSpec((BLK, OUT_C), lambda i: (i, 0)),
        out_shape=jax.ShapeDtypeStruct((N, OUT_C), jnp.float32),
    )(agg_flat, agg_flat, agg_flat, agg_flat, S, qv, R, lw, lb)


def kernel(x, node_types, edge_index_0, edge_index_1, edge_index_2,
           edge_index_3, type_emb, proj_W, proj_b, att_src, att_dst,
           k_lin_W, k_lin_b, q, lin_W, lin_b):
    Wx = proj_W[:IN_C]
    tb = type_emb @ proj_W[IN_C:] + proj_b
    eye8 = jnp.eye(8, dtype=jnp.float32)
    As = jnp.einsum('ehj,hk->hjek', att_src, eye8).reshape(HID, 32)
    Ad = jnp.einsum('ehj,hk->hjek', att_dst, eye8).reshape(HID, 32)
    A = jnp.concatenate([As, Ad], axis=1)
    R = jnp.repeat(eye8, 8, axis=1)

    tsrc, tdst = _phase_a(x, node_types, Wx, tb, A)
    tsrc_flat = tsrc.reshape(NUM_ET * N, 72)
    tdst_flat = tdst.reshape(NUM_ET * N, 16)

    eis = (edge_index_0, edge_index_1, edge_index_2, edge_index_3)
    pad = jnp.zeros((CHUNK,), jnp.int32)
    edges_cat = jnp.concatenate(
        [ei.astype(jnp.int32).reshape(-1) for ei in eis] + [pad])
    agg_flat = _phase_b_sc(tsrc_flat, tdst_flat, edges_cat)

    S = _phase_c1(agg_flat, k_lin_W, k_lin_b.reshape(1, HID), R)
    out = _phase_c2(agg_flat, S, q.reshape(HID, 1), R, lin_W,
                    lin_b.reshape(1, OUT_C))
    return out
